# baseline (speedup 1.0000x reference)
import functools

import jax
import jax.numpy as jnp
from jax import lax
from jax.experimental import pallas as pl
from jax.experimental.pallas import tpu as pltpu, tpu_sc as plsc

NUM_CORES = 2
NUM_SUBCORES = 16
LANES = 16
NW = NUM_CORES * NUM_SUBCORES
BATCH = 16384
EMBED_DIM = 32
B_PER_W = BATCH // NW
CHUNK = 128
NCHUNK = B_PER_W // CHUNK
NBLK = B_PER_W // LANES


def _mf_body(ids_u_h, ids_i_h, eu_h, ei_h, bu_h, bi_h, gb_h, out_h,
             idx_u, idx_i, rows_u, rows_i, bu_v, bi_v, gb_v, out_v, sem):
    wid = lax.axis_index("s") * NUM_CORES + lax.axis_index("c")

    pltpu.sync_copy(ids_u_h.at[wid], idx_u)
    pltpu.sync_copy(ids_i_h.at[wid], idx_i)
    pltpu.sync_copy(gb_h, gb_v.at[pl.ds(0, 1)])

    copies = []
    for j in range(NCHUNK):
        dst = pl.ds(j * CHUNK, CHUNK)
        copies.append(pltpu.async_copy(eu_h.at[idx_u.at[j]], rows_u.at[dst], sem))
        copies.append(pltpu.async_copy(ei_h.at[idx_i.at[j]], rows_i.at[dst], sem))
        copies.append(pltpu.async_copy(bu_h.at[idx_u.at[j]], bu_v.at[dst], sem))
        copies.append(pltpu.async_copy(bi_h.at[idx_i.at[j]], bi_v.at[dst], sem))
    for cp in copies:
        cp.wait()

    gb = gb_v[:][0]

    def blk_body(blk, carry):
        row = blk * LANES + lax.iota(jnp.int32, LANES)
        rows16 = pl.ds(blk * LANES, LANES)
        acc = bu_v[rows16] + bi_v[rows16] + gb
        for d in range(EMBED_DIM):
            dv = jnp.full((LANES,), d, jnp.int32)
            acc = acc + (plsc.load_gather(rows_u, [row, dv])
                         * plsc.load_gather(rows_i, [row, dv]))
        out_v[pl.ds(blk * LANES, LANES)] = acc
        return carry

    lax.fori_loop(0, NBLK, blk_body, 0)

    pltpu.sync_copy(out_v, out_h.at[pl.ds(wid * B_PER_W, B_PER_W)])


@jax.jit
def _mf(ids_u3, ids_i3, eu, ei, bu, bi, gb):
    mesh = plsc.VectorSubcoreMesh(core_axis_name="c", subcore_axis_name="s",
                                  num_cores=NUM_CORES, num_subcores=NUM_SUBCORES)
    return pl.kernel(
        _mf_body,
        out_type=jax.ShapeDtypeStruct((BATCH,), jnp.float32),
        mesh=mesh,
        scratch_types=[
            pltpu.VMEM((NCHUNK, CHUNK), jnp.int32),
            pltpu.VMEM((NCHUNK, CHUNK), jnp.int32),
            pltpu.VMEM((B_PER_W, EMBED_DIM), jnp.float32),
            pltpu.VMEM((B_PER_W, EMBED_DIM), jnp.float32),
            pltpu.VMEM((B_PER_W,), jnp.float32),
            pltpu.VMEM((B_PER_W,), jnp.float32),
            pltpu.VMEM((LANES,), jnp.float32),
            pltpu.VMEM((B_PER_W,), jnp.float32),
            pltpu.SemaphoreType.DMA,
        ],
        compiler_params=pltpu.CompilerParams(needs_layout_passes=False,
                                             use_tc_tiling_on_sc=False),
    )(ids_u3, ids_i3, eu, ei, bu, bi, gb)


def kernel(ids, embedding_users, embedding_items, bias_users, bias_items, global_bias):
    ids_u3 = ids[:, 0].reshape(NW, NCHUNK, CHUNK)
    ids_i3 = ids[:, 1].reshape(NW, NCHUNK, CHUNK)
    return _mf(ids_u3, ids_i3, embedding_users, embedding_items,
               bias_users.reshape(-1), bias_items.reshape(-1), global_bias)
